# Initial kernel scaffold; baseline (speedup 1.0000x reference)
import jax
import jax.numpy as jnp
from jax.experimental import pallas as pl


def kernel(node_feat, edge_index, gin_w1, gin_b1, gin_w2, gin_b2, att_w1, att_w2):
    raise NotImplementedError("write your pallas kernel here")


# speedup vs baseline: 2.7742x; 2.7742x over previous
import functools

import jax
import jax.numpy as jnp
from jax import lax
from jax.experimental import pallas as pl
from jax.experimental.pallas import tpu as pltpu
from jax.experimental.pallas import tpu_sc as plsc

N, R, D, A, E = 10000, 3, 128, 64, 320000

NC = 2
NS = 16
NW = NC * NS
PER_W = E // NW
CK = 128
NCH = -(-PER_W // CK)
PER_W_PAD = 80 * CK
NPAD = 10112
ROWS_PER_TILE = NPAD // NS


def _sc_body(x0, x1, x2, srcp, dstp, out, agg, src_b, dst_b, rows_b, sem):
    c = lax.axis_index("c")
    s = lax.axis_index("s")
    wid = c * NS + s
    base = s * ROWS_PER_TILE

    for r in range(R):
        x_hbm = (x0, x1, x2)[r]
        def _zrow(i, _):
            def _zcol(k, _):
                rows_b[i, pl.ds(k * 16, 16)] = jnp.zeros((16,), jnp.float32)
                return 0
            return lax.fori_loop(0, D // 16, _zcol, 0)
        lax.fori_loop(0, CK, _zrow, 0)
        nfull = ROWS_PER_TILE // CK
        for k in range(nfull):
            pltpu.sync_copy(rows_b, agg.at[pl.ds(base + k * CK, CK)])
        rem = ROWS_PER_TILE - nfull * CK
        if rem:
            pltpu.sync_copy(rows_b.at[pl.ds(0, rem)],
                            agg.at[pl.ds(base + nfull * CK, rem)])
        plsc.subcore_barrier()

        rb = (r * NW + wid) * 80
        pltpu.sync_copy(srcp.at[pl.ds(rb, 80)], src_b)
        pltpu.sync_copy(dstp.at[pl.ds(rb, 80)], dst_b)

        def _chunk(j, _):
            pltpu.async_copy(x_hbm.at[src_b.at[j]], rows_b, sem).wait()
            pltpu.sync_copy(rows_b, agg.at[dst_b.at[j]], add=True)
            return 0
        lax.fori_loop(0, 80, _chunk, 0)
        plsc.subcore_barrier()

        orow = (r * NC + c) * NPAD + base
        pltpu.sync_copy(agg.at[pl.ds(base, ROWS_PER_TILE)],
                        out.at[pl.ds(orow, ROWS_PER_TILE)])
        plsc.subcore_barrier()


def _make_sc_call():
    mesh = plsc.VectorSubcoreMesh(core_axis_name="c", subcore_axis_name="s")
    return functools.partial(
        pl.kernel, mesh=mesh,
        out_type=jax.ShapeDtypeStruct((R * NC * NPAD, D), jnp.float32),
        scratch_types=[
            pltpu.VMEM_SHARED((NPAD, D), jnp.float32),
            pltpu.VMEM((80, CK), jnp.int32),
            pltpu.VMEM((80, CK), jnp.int32),
            pltpu.VMEM((CK, D), jnp.float32),
            pltpu.SemaphoreType.DMA,
        ],
    )(_sc_body)


def _tc_body(xT_ref, p_ref, w1_ref, b1_ref, w2_ref, b2_ref, aw1_ref, aw2_ref,
             out_ref):
    hs = []
    lgs = []
    for r in range(R):
        pre = xT_ref[r] + p_ref[r, 0] + p_ref[r, 1]
        h1 = jnp.maximum(
            jnp.dot(pre, w1_ref[...], preferred_element_type=jnp.float32)
            + b1_ref[...], 0.0)
        h = jnp.maximum(
            jnp.dot(h1, w2_ref[...], preferred_element_type=jnp.float32)
            + b2_ref[...], 0.0)
        t = jnp.tanh(jnp.dot(h, aw1_ref[r], preferred_element_type=jnp.float32))
        lg = jnp.dot(t, aw2_ref[r], preferred_element_type=jnp.float32)
        hs.append(h)
        lgs.append(lg)
    m = jnp.maximum(jnp.maximum(lgs[0], lgs[1]), lgs[2])
    es = [jnp.exp(lg - m) for lg in lgs]
    den = es[0] + es[1] + es[2]
    for i in range(R):
        s_i = es[i] / den
        o = s_i[:, 0:1] * hs[0]
        for j in range(1, R):
            o = o + s_i[:, j:j + 1] * hs[j]
        out_ref[:, i, :] = o


def _dense(xT, partials, gin_w1, gin_b1, gin_w2, gin_b2, att_w1, att_w2):
    B = 1000
    grid = (N // B,)
    return pl.pallas_call(
        _tc_body,
        grid=grid,
        in_specs=[
            pl.BlockSpec((R, B, D), lambda i: (0, i, 0)),
            pl.BlockSpec((R, NC, B, D), lambda i: (0, 0, i, 0)),
            pl.BlockSpec((D, D), lambda i: (0, 0)),
            pl.BlockSpec((1, D), lambda i: (0, 0)),
            pl.BlockSpec((D, D), lambda i: (0, 0)),
            pl.BlockSpec((1, D), lambda i: (0, 0)),
            pl.BlockSpec((R, D, A), lambda i: (0, 0, 0)),
            pl.BlockSpec((R, A, R), lambda i: (0, 0, 0)),
        ],
        out_specs=pl.BlockSpec((B, R, D), lambda i: (i, 0, 0)),
        out_shape=jax.ShapeDtypeStruct((N, R, D), jnp.float32),
    )(xT, partials, gin_w1, gin_b1.reshape(1, D), gin_w2,
      gin_b2.reshape(1, D), att_w1, att_w2)


def kernel(node_feat, edge_index, gin_w1, gin_b1, gin_w2, gin_b2, att_w1,
           att_w2):
    xT = jnp.transpose(node_feat, (1, 0, 2))
    src = edge_index[:, 0, :].reshape(R, NW, PER_W)
    dst = edge_index[:, 1, :].reshape(R, NW, PER_W)
    pad = PER_W_PAD - PER_W
    srcp = jnp.pad(src, ((0, 0), (0, 0), (0, pad)))
    dstp = jnp.pad(dst, ((0, 0), (0, 0), (0, pad)),
                   constant_values=N)
    srcp = srcp.reshape(R * NW * 80, CK)
    dstp = dstp.reshape(R * NW * 80, CK)

    sc = _make_sc_call()
    part = sc(xT[0], xT[1], xT[2], srcp, dstp)
    part = part.reshape(R, NC, NPAD, D)

    return _dense(xT, part, gin_w1, gin_b1, gin_w2, gin_b2, att_w1, att_w2)


# speedup vs baseline: 3.1594x; 1.1389x over previous
import functools

import jax
import jax.numpy as jnp
from jax import lax
from jax.experimental import pallas as pl
from jax.experimental.pallas import tpu as pltpu
from jax.experimental.pallas import tpu_sc as plsc

N, R, D, A, E = 10000, 3, 128, 64, 320000

NC = 2
NS = 16
NW = NC * NS
PER_W = E // NW
CK = 128
NCH = -(-PER_W // CK)
PER_W_PAD = 80 * CK
NPAD = 10112
ROWS_PER_TILE = NPAD // NS


def _sc_body(x0, x1, x2, srcp, dstp, out, agg, src_b, dst_b, rows0, rows1,
             sem0, sem1):
    c = lax.axis_index("c")
    s = lax.axis_index("s")
    wid = c * NS + s
    base = s * ROWS_PER_TILE

    for r in range(R):
        x_hbm = (x0, x1, x2)[r]
        def _zrow(i, _):
            def _zcol(k, _):
                rows0[i, pl.ds(k * 16, 16)] = jnp.zeros((16,), jnp.float32)
                return 0
            return lax.fori_loop(0, D // 16, _zcol, 0)
        lax.fori_loop(0, CK, _zrow, 0)
        nfull = ROWS_PER_TILE // CK
        for k in range(nfull):
            pltpu.sync_copy(rows0, agg.at[pl.ds(base + k * CK, CK)])
        rem = ROWS_PER_TILE - nfull * CK
        if rem:
            pltpu.sync_copy(rows0.at[pl.ds(0, rem)],
                            agg.at[pl.ds(base + nfull * CK, rem)])
        plsc.subcore_barrier()

        rb = (r * NW + wid) * 80
        for half in range(2):
            pltpu.sync_copy(srcp.at[pl.ds(rb + half * 40, 40)], src_b)
            pltpu.sync_copy(dstp.at[pl.ds(rb + half * 40, 40)], dst_b)

            pltpu.async_copy(x_hbm.at[src_b.at[0]], rows0, sem0)

            def _pair(j, _):
                pltpu.async_copy(x_hbm.at[src_b.at[2 * j + 1]], rows1, sem1)
                pltpu.make_async_copy(x_hbm.at[src_b.at[2 * j]], rows0,
                                      sem0).wait()
                pltpu.sync_copy(rows0, agg.at[dst_b.at[2 * j]], add=True)

                @pl.when(j < 19)
                def _():
                    pltpu.async_copy(x_hbm.at[src_b.at[2 * j + 2]], rows0,
                                     sem0)
                pltpu.make_async_copy(x_hbm.at[src_b.at[2 * j + 1]], rows1,
                                      sem1).wait()
                pltpu.sync_copy(rows1, agg.at[dst_b.at[2 * j + 1]], add=True)
                return 0
            lax.fori_loop(0, 20, _pair, 0)
        plsc.subcore_barrier()

        orow = (r * NC + c) * NPAD + base
        pltpu.sync_copy(agg.at[pl.ds(base, ROWS_PER_TILE)],
                        out.at[pl.ds(orow, ROWS_PER_TILE)])
        plsc.subcore_barrier()


def _make_sc_call():
    mesh = plsc.VectorSubcoreMesh(core_axis_name="c", subcore_axis_name="s")
    return functools.partial(
        pl.kernel, mesh=mesh,
        out_type=jax.ShapeDtypeStruct((R * NC * NPAD, D), jnp.float32),
        scratch_types=[
            pltpu.VMEM_SHARED((NPAD, D), jnp.float32),
            pltpu.VMEM((40, CK), jnp.int32),
            pltpu.VMEM((40, CK), jnp.int32),
            pltpu.VMEM((CK, D), jnp.float32),
            pltpu.VMEM((CK, D), jnp.float32),
            pltpu.SemaphoreType.DMA,
            pltpu.SemaphoreType.DMA,
        ],
    )(_sc_body)


def _tc_body(xT_ref, p_ref, w1_ref, b1_ref, w2_ref, b2_ref, aw1_ref, aw2_ref,
             out_ref):
    hs = []
    lgs = []
    for r in range(R):
        pre = xT_ref[r] + p_ref[r, 0] + p_ref[r, 1]
        h1 = jnp.maximum(
            jnp.dot(pre, w1_ref[...], preferred_element_type=jnp.float32)
            + b1_ref[...], 0.0)
        h = jnp.maximum(
            jnp.dot(h1, w2_ref[...], preferred_element_type=jnp.float32)
            + b2_ref[...], 0.0)
        t = jnp.tanh(jnp.dot(h, aw1_ref[r], preferred_element_type=jnp.float32))
        lg = jnp.dot(t, aw2_ref[r], preferred_element_type=jnp.float32)
        hs.append(h)
        lgs.append(lg)
    m = jnp.maximum(jnp.maximum(lgs[0], lgs[1]), lgs[2])
    es = [jnp.exp(lg - m) for lg in lgs]
    den = es[0] + es[1] + es[2]
    for i in range(R):
        s_i = es[i] / den
        o = s_i[:, 0:1] * hs[0]
        for j in range(1, R):
            o = o + s_i[:, j:j + 1] * hs[j]
        out_ref[:, i, :] = o


def _dense(xT, partials, gin_w1, gin_b1, gin_w2, gin_b2, att_w1, att_w2):
    B = 1000
    grid = (N // B,)
    return pl.pallas_call(
        _tc_body,
        grid=grid,
        in_specs=[
            pl.BlockSpec((R, B, D), lambda i: (0, i, 0)),
            pl.BlockSpec((R, NC, B, D), lambda i: (0, 0, i, 0)),
            pl.BlockSpec((D, D), lambda i: (0, 0)),
            pl.BlockSpec((1, D), lambda i: (0, 0)),
            pl.BlockSpec((D, D), lambda i: (0, 0)),
            pl.BlockSpec((1, D), lambda i: (0, 0)),
            pl.BlockSpec((R, D, A), lambda i: (0, 0, 0)),
            pl.BlockSpec((R, A, R), lambda i: (0, 0, 0)),
        ],
        out_specs=pl.BlockSpec((B, R, D), lambda i: (i, 0, 0)),
        out_shape=jax.ShapeDtypeStruct((N, R, D), jnp.float32),
    )(xT, partials, gin_w1, gin_b1.reshape(1, D), gin_w2,
      gin_b2.reshape(1, D), att_w1, att_w2)


def kernel(node_feat, edge_index, gin_w1, gin_b1, gin_w2, gin_b2, att_w1,
           att_w2):
    xT = jnp.transpose(node_feat, (1, 0, 2))
    src = edge_index[:, 0, :].reshape(R, NW, PER_W)
    dst = edge_index[:, 1, :].reshape(R, NW, PER_W)
    pad = PER_W_PAD - PER_W
    srcp = jnp.pad(src, ((0, 0), (0, 0), (0, pad)))
    dstp = jnp.pad(dst, ((0, 0), (0, 0), (0, pad)),
                   constant_values=N)
    srcp = srcp.reshape(R * NW * 80, CK)
    dstp = dstp.reshape(R * NW * 80, CK)

    sc = _make_sc_call()
    part = sc(xT[0], xT[1], xT[2], srcp, dstp)
    part = part.reshape(R, NC, NPAD, D)

    return _dense(xT, part, gin_w1, gin_b1, gin_w2, gin_b2, att_w1, att_w2)
